# baseline (speedup 1.0000x reference)
import functools
import math

import jax
import jax.numpy as jnp
from jax import lax
from jax.experimental import pallas as pl
from jax.experimental.pallas import tpu as pltpu
from jax.experimental.pallas import tpu_sc as plsc

_CH = 1024
_SUB = 128
_CH2 = 512


def _mesh():
    return plsc.VectorSubcoreMesh(core_axis_name="c", subcore_axis_name="s")


@functools.cache
def _make_idx_kernel(n_pad, gd):
    mesh = _mesh()
    nw = mesh.num_cores * mesh.num_subcores
    nchunks = n_pad // _CH
    t_iters = math.ceil(nchunks / nw)
    s_r = gd * gd * gd
    s_h = gd * gd
    s_k = gd

    @functools.partial(
        pl.kernel,
        out_type=jax.ShapeDtypeStruct((n_pad,), jnp.int32),
        mesh=mesh,
        compiler_params=pltpu.CompilerParams(
            needs_layout_passes=False, use_tc_tiling_on_sc=False),
        scratch_types=[
            pltpu.VMEM((_CH,), jnp.int32),
            pltpu.VMEM((_CH,), jnp.int32),
            pltpu.VMEM((_CH,), jnp.int32),
            pltpu.VMEM((_CH,), jnp.int32),
            pltpu.VMEM((_CH,), jnp.int32),
            pltpu.VMEM((_CH,), jnp.int32),
            pltpu.SemaphoreType.DMA,
        ],
    )
    def idx_kernel(rasu_hbm, h_hbm, k_hbm, l_hbm, grid_hbm, out_hbm,
                   hbuf, kbuf, lbuf, rbuf, flatb, idxb, sem):
        wid = lax.axis_index("s") * mesh.num_cores + lax.axis_index("c")

        def chunk_body(t, carry):
            c = t * nw + wid

            @pl.when(c < nchunks)
            def _():
                base = c * _CH
                pltpu.sync_copy(h_hbm.at[pl.ds(base, _CH)], hbuf)
                pltpu.sync_copy(k_hbm.at[pl.ds(base, _CH)], kbuf)
                pltpu.sync_copy(l_hbm.at[pl.ds(base, _CH)], lbuf)
                pltpu.sync_copy(rasu_hbm.at[pl.ds(base, _CH)], rbuf)

                def group_body(g, gcarry):
                    sl = pl.ds(g * 16, 16)
                    flatb[sl] = (rbuf[sl] * s_r + hbuf[sl] * s_h
                                 + kbuf[sl] * s_k + lbuf[sl])
                    return gcarry

                lax.fori_loop(0, _CH // 16, group_body, 0)

                descs = [
                    pltpu.async_copy(
                        grid_hbm.at[flatb.at[pl.ds(j * _SUB, _SUB)]],
                        idxb.at[pl.ds(j * _SUB, _SUB)], sem)
                    for j in range(_CH // _SUB)
                ]
                for de in descs:
                    de.wait()
                pltpu.sync_copy(idxb, out_hbm.at[pl.ds(base, _CH)])
            return carry

        lax.fori_loop(0, t_iters, chunk_body, 0)

    return idx_kernel


@functools.cache
def _make_row_kernel(n_out, n_pad, d, rac2):
    mesh = _mesh()
    nw = mesh.num_cores * mesh.num_subcores
    n_til = math.ceil(n_out / 128) * 128
    nchunks = math.ceil(n_til / _CH2)
    t_iters = math.ceil(nchunks / nw)
    tail_c = nchunks - 1
    tail_n = n_til - tail_c * _CH2

    @functools.partial(
        pl.kernel,
        out_type=jax.ShapeDtypeStruct((d, n_til), jnp.float32),
        mesh=mesh,
        compiler_params=pltpu.CompilerParams(
            needs_layout_passes=False, use_tc_tiling_on_sc=True),
        scratch_types=[
            pltpu.VMEM((_CH2,), jnp.int32),
            pltpu.VMEM((_CH2,), jnp.int32),
            pltpu.VMEM((_CH2, 128), jnp.float32),
            pltpu.VMEM((d, _CH2), jnp.float32),
            pltpu.SemaphoreType.DMA,
        ],
    )
    def row_kernel(src_hbm, idx_hbm, out_hbm, idxc, rowix, fetched, outt,
                   sem):
        wid = lax.axis_index("s") * mesh.num_cores + lax.axis_index("c")
        lanes = lax.iota(jnp.int32, 16)

        def chunk_body(t, carry):
            c = t * nw + wid

            @pl.when(c < nchunks)
            def _():
                base = c * _CH2
                pltpu.sync_copy(idx_hbm.at[pl.ds(base, _CH2)], idxc)

                def rg_body(g, gcarry):
                    sl = pl.ds(g * 16, 16)
                    rowix[sl] = idxc[sl] >> 1
                    return gcarry

                lax.fori_loop(0, _CH2 // 16, rg_body, 0)

                descs = [
                    pltpu.async_copy(
                        src_hbm.at[rowix.at[pl.ds(j * _SUB, _SUB)]],
                        fetched.at[pl.ds(j * _SUB, _SUB)], sem)
                    for j in range(_CH2 // _SUB)
                ]
                for de in descs:
                    de.wait()

                def tr_body(g, gcarry):
                    rsl = pl.ds(g * 16, 16)
                    rows16 = g * 16 + lanes
                    cols0 = (idxc[rsl] & 1) * (d)

                    def f_body(f, fcarry):
                        outt[f, rsl] = plsc.load_gather(
                            fetched, [rows16, cols0 + f])
                        return fcarry

                    lax.fori_loop(0, d, f_body, 0)
                    return gcarry

                lax.fori_loop(0, _CH2 // 16, tr_body, 0)

                if tail_n == _CH2:
                    pltpu.sync_copy(outt, out_hbm.at[:, pl.ds(base, _CH2)])
                else:
                    @pl.when(c != tail_c)
                    def _():
                        pltpu.sync_copy(outt,
                                        out_hbm.at[:, pl.ds(base, _CH2)])

                    @pl.when(c == tail_c)
                    def _():
                        pltpu.sync_copy(outt.at[:, pl.ds(0, tail_n)],
                                        out_hbm.at[:, pl.ds(base, tail_n)])
            return carry

        lax.fori_loop(0, t_iters, chunk_body, 0)

    return row_kernel


def kernel(source, rasu_id, H, reflection_id_grid):
    n = rasu_id.shape[0]
    rac, d = source.shape
    gd = reflection_id_grid.shape[1]
    n_pad = math.ceil(n / _CH) * _CH
    pad = n_pad - n
    H = H.astype(jnp.int32)
    rasu = jnp.pad(rasu_id.astype(jnp.int32), (0, pad))
    h1 = jnp.pad(H[:, 0], (0, pad))
    k1 = jnp.pad(H[:, 1], (0, pad))
    l1 = jnp.pad(H[:, 2], (0, pad))
    grid1d = reflection_id_grid.reshape(-1)
    src128 = source.reshape(rac // 2, 2 * d)
    idx1 = _make_idx_kernel(n_pad, gd)(rasu, h1, k1, l1, grid1d)
    outt = _make_row_kernel(n, n_pad, d, rac // 2)(src128, idx1)
    return outt[:, :n].T


# speedup vs baseline: 1.8961x; 1.8961x over previous
import functools
import math

import jax
import jax.numpy as jnp
from jax import lax
from jax.experimental import pallas as pl
from jax.experimental.pallas import tpu as pltpu
from jax.experimental.pallas import tpu_sc as plsc

_CH = 1024
_SUB = 128
_NSUB = _CH // _SUB


@functools.cache
def _make_sc_gather(n_out, d, gd):
    mesh = plsc.VectorSubcoreMesh(core_axis_name="c", subcore_axis_name="s")
    nw = mesh.num_cores * mesh.num_subcores
    nchunks = math.ceil(n_out / _CH)
    t_iters = math.ceil(nchunks / nw)
    s_r = gd * gd * gd
    s_h = gd * gd
    s_k = gd
    tail_c = (n_out - 1) // _CH
    tail_n = n_out - tail_c * _CH

    @functools.partial(
        pl.kernel,
        out_type=jax.ShapeDtypeStruct((n_out, d), jnp.float32),
        mesh=mesh,
        compiler_params=pltpu.CompilerParams(
            needs_layout_passes=False, use_tc_tiling_on_sc=False),
        scratch_types=[
            pltpu.VMEM((_CH,), jnp.int32),
            pltpu.VMEM((_CH,), jnp.int32),
            pltpu.VMEM((_CH,), jnp.int32),
            pltpu.VMEM((_CH,), jnp.int32),
            pltpu.VMEM((_CH,), jnp.int32),
            pltpu.VMEM((_CH,), jnp.int32),
            pltpu.VMEM((_CH, d), jnp.float32),
            pltpu.SemaphoreType.DMA,
        ],
    )
    def gather_kernel(src_hbm, rasu_hbm, h_hbm, k_hbm, l_hbm, grid_hbm,
                      out_hbm, hbuf, kbuf, lbuf, rbuf, flatb, idxb, rows,
                      sem):
        wid = lax.axis_index("s") * mesh.num_cores + lax.axis_index("c")

        def chunk_body(t, carry):
            c = t * nw + wid

            @pl.when(c < nchunks)
            def _():
                base = c * _CH
                pltpu.sync_copy(h_hbm.at[pl.ds(base, _CH)], hbuf)
                pltpu.sync_copy(k_hbm.at[pl.ds(base, _CH)], kbuf)
                pltpu.sync_copy(l_hbm.at[pl.ds(base, _CH)], lbuf)
                pltpu.sync_copy(rasu_hbm.at[pl.ds(base, _CH)], rbuf)

                def group_body(g, gcarry):
                    i0 = g * 16
                    sl = pl.ds(i0, 16)
                    flatb[sl] = (rbuf[sl] * s_r + hbuf[sl] * s_h
                                 + kbuf[sl] * s_k + lbuf[sl])
                    return gcarry

                lax.fori_loop(0, _CH // 16, group_body, 0)

                descs = [
                    pltpu.async_copy(
                        grid_hbm.at[flatb.at[pl.ds(j * _SUB, _SUB)]],
                        idxb.at[pl.ds(j * _SUB, _SUB)], sem)
                    for j in range(_NSUB)
                ]
                for de in descs:
                    de.wait()
                descs = [
                    pltpu.async_copy(
                        src_hbm.at[idxb.at[pl.ds(j * _SUB, _SUB)]],
                        rows.at[pl.ds(j * _SUB, _SUB)], sem)
                    for j in range(_NSUB)
                ]
                for de in descs:
                    de.wait()

                if tail_n == _CH:
                    pltpu.sync_copy(rows, out_hbm.at[pl.ds(base, _CH)])
                else:
                    @pl.when(c != tail_c)
                    def _():
                        pltpu.sync_copy(rows, out_hbm.at[pl.ds(base, _CH)])

                    @pl.when(c == tail_c)
                    def _():
                        pltpu.sync_copy(rows.at[pl.ds(0, tail_n)],
                                        out_hbm.at[pl.ds(base, tail_n)])
            return carry

        lax.fori_loop(0, t_iters, chunk_body, 0)

    return gather_kernel


def kernel(source, rasu_id, H, reflection_id_grid):
    n = rasu_id.shape[0]
    d = source.shape[1]
    gd = reflection_id_grid.shape[1]
    n_pad = math.ceil(n / _CH) * _CH
    pad = n_pad - n
    H = H.astype(jnp.int32)
    rasu = jnp.pad(rasu_id.astype(jnp.int32), (0, pad))
    h1 = jnp.pad(H[:, 0], (0, pad))
    k1 = jnp.pad(H[:, 1], (0, pad))
    l1 = jnp.pad(H[:, 2], (0, pad))
    grid1d = reflection_id_grid.reshape(-1)
    fn = _make_sc_gather(n, d, gd)
    return fn(source, rasu, h1, k1, l1, grid1d)


# speedup vs baseline: 2.0809x; 1.0974x over previous
import functools
import math

import jax
import jax.numpy as jnp
from jax import lax
from jax.experimental import pallas as pl
from jax.experimental.pallas import tpu as pltpu
from jax.experimental.pallas import tpu_sc as plsc

_CH = 1024
_SUB = 128
_NSUB = _CH // _SUB


def _mesh():
    return plsc.VectorSubcoreMesh(core_axis_name="c", subcore_axis_name="s")


@functools.cache
def _make_idx_kernel(n_pad, gd):
    mesh = _mesh()
    nw = mesh.num_cores * mesh.num_subcores
    nchunks = n_pad // _CH
    t_iters = math.ceil(nchunks / nw)
    s_r = gd * gd * gd
    s_h = gd * gd
    s_k = gd

    @functools.partial(
        pl.kernel,
        out_type=jax.ShapeDtypeStruct((n_pad,), jnp.int32),
        mesh=mesh,
        compiler_params=pltpu.CompilerParams(
            needs_layout_passes=False, use_tc_tiling_on_sc=False),
        scratch_types=[
            pltpu.VMEM((_CH,), jnp.int32),
            pltpu.VMEM((_CH,), jnp.int32),
            pltpu.VMEM((_CH,), jnp.int32),
            pltpu.VMEM((_CH,), jnp.int32),
            pltpu.VMEM((_CH,), jnp.int32),
            pltpu.VMEM((_CH,), jnp.int32),
            pltpu.SemaphoreType.DMA,
        ],
    )
    def idx_kernel(rasu_hbm, h_hbm, k_hbm, l_hbm, grid_hbm, out_hbm,
                   hbuf, kbuf, lbuf, rbuf, flatb, idxb, sem):
        wid = lax.axis_index("s") * mesh.num_cores + lax.axis_index("c")

        def chunk_body(t, carry):
            c = t * nw + wid

            @pl.when(c < nchunks)
            def _():
                base = c * _CH
                pltpu.sync_copy(h_hbm.at[pl.ds(base, _CH)], hbuf)
                pltpu.sync_copy(k_hbm.at[pl.ds(base, _CH)], kbuf)
                pltpu.sync_copy(l_hbm.at[pl.ds(base, _CH)], lbuf)
                pltpu.sync_copy(rasu_hbm.at[pl.ds(base, _CH)], rbuf)

                def group_body(g, gcarry):
                    sl = pl.ds(g * 16, 16)
                    flatb[sl] = (rbuf[sl] * s_r + hbuf[sl] * s_h
                                 + kbuf[sl] * s_k + lbuf[sl])
                    return gcarry

                lax.fori_loop(0, _CH // 16, group_body, 0)

                descs = [
                    pltpu.async_copy(
                        grid_hbm.at[flatb.at[pl.ds(j * _SUB, _SUB)]],
                        idxb.at[pl.ds(j * _SUB, _SUB)], sem)
                    for j in range(_NSUB)
                ]
                for de in descs:
                    de.wait()
                pltpu.sync_copy(idxb, out_hbm.at[pl.ds(base, _CH)])
            return carry

        lax.fori_loop(0, t_iters, chunk_body, 0)

    return idx_kernel


@functools.cache
def _make_row_kernel(n_out, n_pad, d):
    mesh = _mesh()
    nw = mesh.num_cores * mesh.num_subcores
    nchunks = math.ceil(n_out / _CH)
    t_iters = math.ceil(nchunks / nw)
    tail_c = (n_out - 1) // _CH
    tail_n = n_out - tail_c * _CH
    assert nchunks * _CH <= n_pad

    @functools.partial(
        pl.kernel,
        out_type=jax.ShapeDtypeStruct((n_out, d), jnp.float32),
        mesh=mesh,
        compiler_params=pltpu.CompilerParams(
            needs_layout_passes=False, use_tc_tiling_on_sc=False),
        scratch_types=[
            pltpu.VMEM((_CH,), jnp.int32),
            pltpu.VMEM((_CH, d), jnp.float32),
            pltpu.SemaphoreType.DMA,
        ],
    )
    def row_kernel(src_hbm, idx_hbm, out_hbm, idxb, rows, sem):
        wid = lax.axis_index("s") * mesh.num_cores + lax.axis_index("c")

        def chunk_body(t, carry):
            c = t * nw + wid

            @pl.when(c < nchunks)
            def _():
                base = c * _CH
                pltpu.sync_copy(idx_hbm.at[pl.ds(base, _CH)], idxb)
                descs = [
                    pltpu.async_copy(
                        src_hbm.at[idxb.at[pl.ds(j * _SUB, _SUB)]],
                        rows.at[pl.ds(j * _SUB, _SUB)], sem)
                    for j in range(_NSUB)
                ]
                for de in descs:
                    de.wait()

                if tail_n == _CH:
                    pltpu.sync_copy(rows, out_hbm.at[pl.ds(base, _CH)])
                else:
                    @pl.when(c != tail_c)
                    def _():
                        pltpu.sync_copy(rows, out_hbm.at[pl.ds(base, _CH)])

                    @pl.when(c == tail_c)
                    def _():
                        pltpu.sync_copy(rows.at[pl.ds(0, tail_n)],
                                        out_hbm.at[pl.ds(base, tail_n)])
            return carry

        lax.fori_loop(0, t_iters, chunk_body, 0)

    return row_kernel


def kernel(source, rasu_id, H, reflection_id_grid):
    n = rasu_id.shape[0]
    d = source.shape[1]
    gd = reflection_id_grid.shape[1]
    n_pad = math.ceil(n / _CH) * _CH
    pad = n_pad - n
    H = H.astype(jnp.int32)
    rasu = jnp.pad(rasu_id.astype(jnp.int32), (0, pad))
    h1 = jnp.pad(H[:, 0], (0, pad))
    k1 = jnp.pad(H[:, 1], (0, pad))
    l1 = jnp.pad(H[:, 2], (0, pad))
    grid1d = reflection_id_grid.reshape(-1)
    idx1 = _make_idx_kernel(n_pad, gd)(rasu, h1, k1, l1, grid1d)
    return _make_row_kernel(n, n_pad, d)(source, idx1)
